# baseline (speedup 1.0000x reference)
import functools

import jax
import jax.numpy as jnp
from jax import lax
from jax.experimental import pallas as pl
from jax.experimental.pallas import tpu as pltpu
from jax.experimental.pallas import tpu_sc as plsc

_NC, _NS = 2, 16
_NW = _NC * _NS
_T = 50
_BB = 4096
_D = 64
_BLK = _BB // _NW
_NBUF = 5
_NG = _T // _NBUF

_mesh = plsc.VectorSubcoreMesh(core_axis_name="c", subcore_axis_name="s")


@functools.partial(
    pl.kernel,
    out_type=jax.ShapeDtypeStruct((_T, _D, _BB), jnp.float32),
    mesh=_mesh,
    scratch_types=[
        pltpu.VMEM((_T, _BLK), jnp.int32),
        [pltpu.VMEM((_BLK, _D), jnp.float32) for _ in range(_NBUF)],
        [pltpu.VMEM((_D, _BLK), jnp.float32) for _ in range(_NBUF)],
        [pltpu.SemaphoreType.DMA for _ in range(_NBUF)],
        [pltpu.SemaphoreType.DMA for _ in range(_NBUF)],
    ],
    compiler_params=pltpu.CompilerParams(use_tc_tiling_on_sc=False,
                                         needs_layout_passes=False),
)
def _gather_kernel(idx_hbm, table_hbm, out_hbm, idx_v, gbuf, tbuf, sem_g, sem_w):
    wid = lax.axis_index("s") * _NC + lax.axis_index("c")
    b0 = pl.multiple_of(wid * _BLK, _BLK)
    pltpu.sync_copy(idx_hbm.at[:, pl.ds(b0, _BLK)], idx_v)

    def fire_gather(t, b):
        pltpu.async_copy(table_hbm.at[idx_v.at[t]], gbuf[b], sem_g[b])

    def wait_gather(b):
        pltpu.make_async_copy(table_hbm.at[idx_v.at[0]], gbuf[b], sem_g[b]).wait()

    def transpose(b):
        @pl.loop(0, _D)
        def _col(e):
            col = jnp.full((16,), e, dtype=jnp.int32)
            for g in range(_BLK // 16):
                rows = jnp.arange(16, dtype=jnp.int32) + (g * 16)
                v = plsc.load_gather(gbuf[b], [rows, col])
                tbuf[b][e, pl.ds(g * 16, 16)] = v

    def fire_writeback(t, b):
        pltpu.async_copy(tbuf[b], out_hbm.at[t, :, pl.ds(b0, _BLK)], sem_w[b])

    def wait_writeback(b):
        pltpu.make_async_copy(tbuf[b], out_hbm.at[0, :, pl.ds(0, _BLK)],
                              sem_w[b]).wait()

    for b in range(_NBUF):
        fire_gather(b, b)
    for b in range(_NBUF):
        wait_gather(b)
        transpose(b)
        fire_writeback(b, b)
        fire_gather(b + _NBUF, b)

    @pl.loop(1, _NG - 1)
    def _group(gi):
        t0 = gi * _NBUF
        for b in range(_NBUF):
            wait_gather(b)
            wait_writeback(b)
            transpose(b)
            fire_writeback(t0 + b, b)
            fire_gather(t0 + b + _NBUF, b)

    t0 = (_NG - 1) * _NBUF
    for b in range(_NBUF):
        wait_gather(b)
        wait_writeback(b)
        transpose(b)
        fire_writeback(t0 + b, b)
    for b in range(_NBUF):
        wait_writeback(b)


def kernel(input_ids, weight):
    idx_t = input_ids.T.astype(jnp.int32)
    out_phys = _gather_kernel(idx_t, weight)
    return out_phys.transpose(2, 0, 1)


# speedup vs baseline: 1.0612x; 1.0612x over previous
import functools

import jax
import jax.numpy as jnp
from jax import lax
from jax.experimental import pallas as pl
from jax.experimental.pallas import tpu as pltpu
from jax.experimental.pallas import tpu_sc as plsc

_NC, _NS = 2, 16
_NW = _NC * _NS
_T = 50
_BB = 4096
_D = 64
_BLK = _BB // _NW
_NBUF = 5
_NG = _T // _NBUF

_mesh = plsc.VectorSubcoreMesh(core_axis_name="c", subcore_axis_name="s")


@functools.partial(
    pl.kernel,
    out_type=jax.ShapeDtypeStruct((_T, _D, _BB), jnp.float32),
    mesh=_mesh,
    scratch_types=[
        pltpu.VMEM((_T, _BLK), jnp.int32),
        [pltpu.VMEM((_BLK, _D), jnp.float32) for _ in range(_NBUF)],
        [pltpu.VMEM((_D, _BLK), jnp.float32) for _ in range(_NBUF)],
        [pltpu.SemaphoreType.DMA for _ in range(_NBUF)],
        [pltpu.SemaphoreType.DMA for _ in range(_NBUF)],
    ],
    compiler_params=pltpu.CompilerParams(use_tc_tiling_on_sc=False,
                                         needs_layout_passes=False),
)
def _gather_kernel(idx_hbm, table_hbm, out_hbm, idx_v, gbuf, tbuf, sem_g, sem_w):
    wid = lax.axis_index("s") * _NC + lax.axis_index("c")
    b0 = pl.multiple_of(wid * _BLK, _BLK)
    pltpu.sync_copy(idx_hbm.at[:, pl.ds(b0, _BLK)], idx_v)

    def fire_gather(t, b):
        pltpu.async_copy(table_hbm.at[idx_v.at[t]], gbuf[b], sem_g[b])

    def wait_gather(b):
        pltpu.make_async_copy(table_hbm.at[idx_v.at[0]], gbuf[b], sem_g[b]).wait()

    row_ids = [jnp.arange(16, dtype=jnp.int32) + (g * 16)
               for g in range(_BLK // 16)]

    def transpose(b):
        @pl.loop(0, _D, unroll=8)
        def _col(e):
            col = jnp.full((16,), e, dtype=jnp.int32)
            vals = [plsc.load_gather(gbuf[b], [rows, col]) for rows in row_ids]
            for g, v in enumerate(vals):
                tbuf[b][e, pl.ds(g * 16, 16)] = v

    def fire_writeback(t, b):
        pltpu.async_copy(tbuf[b], out_hbm.at[t, :, pl.ds(b0, _BLK)], sem_w[b])

    def wait_writeback(b):
        pltpu.make_async_copy(tbuf[b], out_hbm.at[0, :, pl.ds(0, _BLK)],
                              sem_w[b]).wait()

    for b in range(_NBUF):
        fire_gather(b, b)
    for b in range(_NBUF):
        wait_gather(b)
        transpose(b)
        fire_writeback(b, b)
        fire_gather(b + _NBUF, b)

    @pl.loop(1, _NG - 1)
    def _group(gi):
        t0 = gi * _NBUF
        for b in range(_NBUF):
            wait_gather(b)
            wait_writeback(b)
            transpose(b)
            fire_writeback(t0 + b, b)
            fire_gather(t0 + b + _NBUF, b)

    t0 = (_NG - 1) * _NBUF
    for b in range(_NBUF):
        wait_gather(b)
        wait_writeback(b)
        transpose(b)
        fire_writeback(t0 + b, b)
    for b in range(_NBUF):
        wait_writeback(b)


def kernel(input_ids, weight):
    idx_t = input_ids.T.astype(jnp.int32)
    out_phys = _gather_kernel(idx_t, weight)
    return out_phys.transpose(2, 0, 1)


# speedup vs baseline: 1.2989x; 1.2240x over previous
import functools

import jax
import jax.numpy as jnp
from jax import lax
from jax.experimental import pallas as pl
from jax.experimental.pallas import tpu as pltpu
from jax.experimental.pallas import tpu_sc as plsc

_NC, _NS = 2, 16
_NW = _NC * _NS
_T = 50
_BB = 4096
_D = 64
_BLK = _BB // _NW
_NBUF = 5
_NG = _T // _NBUF

_mesh = plsc.VectorSubcoreMesh(core_axis_name="c", subcore_axis_name="s")


@functools.partial(
    pl.kernel,
    out_type=jax.ShapeDtypeStruct((_T, _D, _BB), jnp.float32),
    mesh=_mesh,
    scratch_types=[
        pltpu.VMEM((_T, _BLK), jnp.int32),
        [pltpu.VMEM((_BLK, _D), jnp.float32) for _ in range(_NBUF)],
        [pltpu.VMEM((_D, _BLK + 1), jnp.float32) for _ in range(_NBUF)],
        [pltpu.SemaphoreType.DMA for _ in range(_NBUF)],
        [pltpu.SemaphoreType.DMA for _ in range(_NBUF)],
    ],
    compiler_params=pltpu.CompilerParams(use_tc_tiling_on_sc=False,
                                         needs_layout_passes=False),
)
def _gather_kernel(idx_hbm, table_hbm, out_hbm, idx_v, gbuf, tbuf, sem_g, sem_w):
    wid = lax.axis_index("s") * _NC + lax.axis_index("c")
    b0 = pl.multiple_of(wid * _BLK, _BLK)
    pltpu.sync_copy(idx_hbm.at[:, pl.ds(b0, _BLK)], idx_v)

    def fire_gather(t, b):
        pltpu.async_copy(table_hbm.at[idx_v.at[t]], gbuf[b], sem_g[b])

    def wait_gather(b):
        pltpu.make_async_copy(table_hbm.at[idx_v.at[0]], gbuf[b], sem_g[b]).wait()

    row_ids = [jnp.arange(16, dtype=jnp.int32) + (g * 16)
               for g in range(_D // 16)]

    def transpose(b):
        @pl.loop(0, _BLK, unroll=8)
        def _row(j):
            col = jnp.full((16,), j, dtype=jnp.int32)
            vals = [gbuf[b][j, pl.ds(g * 16, 16)] for g in range(_D // 16)]
            for g, v in enumerate(vals):
                plsc.store_scatter(tbuf[b], [row_ids[g], col], v)

    def fire_writeback(t, b):
        pltpu.async_copy(tbuf[b].at[:, pl.ds(0, _BLK)],
                         out_hbm.at[t, :, pl.ds(b0, _BLK)], sem_w[b])

    def wait_writeback(b):
        pltpu.make_async_copy(tbuf[b].at[:, pl.ds(0, _BLK)],
                              out_hbm.at[0, :, pl.ds(0, _BLK)],
                              sem_w[b]).wait()

    for b in range(_NBUF):
        fire_gather(b, b)
    for b in range(_NBUF):
        wait_gather(b)
        transpose(b)
        fire_writeback(b, b)
        fire_gather(b + _NBUF, b)

    @pl.loop(1, _NG - 1)
    def _group(gi):
        t0 = gi * _NBUF
        for b in range(_NBUF):
            wait_gather(b)
            wait_writeback(b)
            transpose(b)
            fire_writeback(t0 + b, b)
            fire_gather(t0 + b + _NBUF, b)

    t0 = (_NG - 1) * _NBUF
    for b in range(_NBUF):
        wait_gather(b)
        wait_writeback(b)
        transpose(b)
        fire_writeback(t0 + b, b)
    for b in range(_NBUF):
        wait_writeback(b)


def kernel(input_ids, weight):
    idx_t = input_ids.T.astype(jnp.int32)
    out_phys = _gather_kernel(idx_t, weight)
    return out_phys.transpose(2, 0, 1)


# speedup vs baseline: 1.6657x; 1.2823x over previous
import functools

import jax
import jax.numpy as jnp
from jax import lax
from jax.experimental import pallas as pl
from jax.experimental.pallas import tpu as pltpu
from jax.experimental.pallas import tpu_sc as plsc

_NC, _NS = 2, 16
_NW = _NC * _NS
_T = 50
_BB = 4096
_D = 64
_BLK = _BB // _NW
_NBUF = 5
_NG = _T // _NBUF

_mesh = plsc.VectorSubcoreMesh(core_axis_name="c", subcore_axis_name="s")


@functools.partial(
    pl.kernel,
    out_type=jax.ShapeDtypeStruct((_T, _NW, 2, _D, _D), jnp.float32),
    mesh=_mesh,
    scratch_types=[
        pltpu.VMEM((_T, _BLK), jnp.int32),
        [pltpu.VMEM((_BLK, _D), jnp.float32) for _ in range(_NBUF)],
        [pltpu.SemaphoreType.DMA for _ in range(_NBUF)],
        [pltpu.SemaphoreType.DMA for _ in range(_NBUF)],
    ],
    compiler_params=pltpu.CompilerParams(use_tc_tiling_on_sc=False),
)
def _gather_kernel(idx_hbm, table_hbm, rows_hbm, idx_v, gbuf, sem_g, sem_w):
    wid = lax.axis_index("s") * _NC + lax.axis_index("c")
    b0 = pl.multiple_of(wid * _BLK, _BLK)
    pltpu.sync_copy(idx_hbm.at[:, pl.ds(b0, _BLK)], idx_v)

    def fire_gather(t, b):
        pltpu.async_copy(table_hbm.at[idx_v.at[t]], gbuf[b], sem_g[b])

    def wait_gather(b):
        pltpu.make_async_copy(table_hbm.at[idx_v.at[0]], gbuf[b], sem_g[b]).wait()

    def fire_writeback(t, b):
        pltpu.async_copy(gbuf[b].at[pl.ds(0, _D), :], rows_hbm.at[t, wid, 0],
                         sem_w[b])
        pltpu.async_copy(gbuf[b].at[pl.ds(_D, _D), :], rows_hbm.at[t, wid, 1],
                         sem_w[b])

    def wait_writeback(b):
        for _ in range(2):
            pltpu.make_async_copy(gbuf[b].at[pl.ds(0, _D), :],
                                  rows_hbm.at[0, 0, 0], sem_w[b]).wait()

    for b in range(_NBUF):
        fire_gather(b, b)
    for b in range(_NBUF):
        wait_gather(b)
        fire_writeback(b, b)

    @pl.loop(1, _NG)
    def _group(gi):
        t0 = gi * _NBUF
        for b in range(_NBUF):
            wait_writeback(b)
            fire_gather(t0 + b, b)
        for b in range(_NBUF):
            wait_gather(b)
            fire_writeback(t0 + b, b)

    for b in range(_NBUF):
        wait_writeback(b)


@functools.partial(
    pl.kernel,
    out_type=jax.ShapeDtypeStruct((_T, _D // 8, _NW, 8, _BLK), jnp.float32),
    mesh=_mesh,
    scratch_types=[
        [pltpu.VMEM((2, _D, _D), jnp.float32) for _ in range(_NBUF)],
        [pltpu.VMEM((_D // 8, 8, _BLK + 1), jnp.float32) for _ in range(_NBUF)],
        [pltpu.SemaphoreType.DMA for _ in range(_NBUF)],
        [pltpu.SemaphoreType.DMA for _ in range(_NBUF)],
    ],
    compiler_params=pltpu.CompilerParams(use_tc_tiling_on_sc=False,
                                         needs_layout_passes=False),
)
def _retile_kernel(rows_hbm, out_hbm, ibuf, tbuf, sem_i, sem_w):
    wid = lax.axis_index("s") * _NC + lax.axis_index("c")

    def fire_read(t, b):
        pltpu.async_copy(rows_hbm.at[t, wid], ibuf[b], sem_i[b])

    def wait_read(b):
        pltpu.make_async_copy(rows_hbm.at[0, 0], ibuf[b], sem_i[b]).wait()

    lane = jnp.arange(16, dtype=jnp.int32)
    er_ids = [(lane >> 3) + 2 * g for g in range(_D // 16)]
    ei_ids = lane & 7

    def transpose(b):
        @pl.loop(0, _BLK, unroll=8)
        def _row(j):
            col = jnp.full((16,), j, dtype=jnp.int32)
            vals = [ibuf[b][j >> 6, j & (_D - 1), pl.ds(g * 16, 16)]
                    for g in range(_D // 16)]
            for g, v in enumerate(vals):
                plsc.store_scatter(tbuf[b], [er_ids[g], ei_ids, col], v)

    def fire_writeback(t, b):
        pltpu.async_copy(tbuf[b].at[:, :, pl.ds(0, _BLK)],
                         out_hbm.at[t, :, wid, :, :], sem_w[b])

    def wait_writeback(b):
        pltpu.make_async_copy(tbuf[b].at[:, :, pl.ds(0, _BLK)],
                              out_hbm.at[0, :, 0, :, :], sem_w[b]).wait()

    for b in range(_NBUF):
        fire_read(b, b)
    for b in range(_NBUF):
        wait_read(b)
        transpose(b)
        fire_writeback(b, b)
        fire_read(b + _NBUF, b)

    @pl.loop(1, _NG - 1)
    def _group(gi):
        t0 = gi * _NBUF
        for b in range(_NBUF):
            wait_read(b)
            wait_writeback(b)
            transpose(b)
            fire_writeback(t0 + b, b)
            fire_read(t0 + b + _NBUF, b)

    t0 = (_NG - 1) * _NBUF
    for b in range(_NBUF):
        wait_read(b)
        wait_writeback(b)
        transpose(b)
        fire_writeback(t0 + b, b)
    for b in range(_NBUF):
        wait_writeback(b)


_VB = 2048
_NVB = 489
_VPAD = _NVB * _VB


def _transpose_body(x_ref, o_ref):
    x = x_ref[...]
    o_ref[...] = jnp.concatenate(
        [x[:, :_VB // 2].T, x[:, _VB // 2:].T], axis=1)


def _transpose_tc(wt):
    return pl.pallas_call(
        _transpose_body,
        out_shape=jax.ShapeDtypeStruct((_VPAD // 2, 2 * _D), jnp.float32),
        grid=(_NVB,),
        in_specs=[pl.BlockSpec((_D, _VB), lambda i: (0, i))],
        out_specs=pl.BlockSpec((_VB // 2, 2 * _D), lambda i: (i, 0)),
    )(wt)


def kernel(input_ids, weight):
    v = input_ids.T.astype(jnp.int32)
    j = v & (_VB - 1)
    idx_t = (v - j) + 2 * (j & (_VB // 2 - 1)) + (j >> 10)
    table = _transpose_tc(weight.T).reshape(_VPAD, _D)
    rows = _gather_kernel(idx_t, table)
    out5 = _retile_kernel(rows)
    out = out5.transpose(2, 4, 0, 1, 3).reshape(_BB, _T, _D)
    return out


# speedup vs baseline: 2.0967x; 1.2588x over previous
import functools

import jax
import jax.numpy as jnp
from jax import lax
from jax.experimental import pallas as pl
from jax.experimental.pallas import tpu as pltpu
from jax.experimental.pallas import tpu_sc as plsc

_NC, _NS = 2, 16
_NW = _NC * _NS
_T = 50
_BB = 4096
_D = 64
_BLK = _BB // _NW
_NBUF = 5
_NG = _T // _NBUF

_mesh = plsc.VectorSubcoreMesh(core_axis_name="c", subcore_axis_name="s")


@functools.partial(
    pl.kernel,
    out_type=jax.ShapeDtypeStruct((_T, _NW, 2, _D, _D), jnp.float32),
    mesh=_mesh,
    scratch_types=[
        pltpu.VMEM((_T, _BLK), jnp.int32),
        [pltpu.VMEM((_BLK, _D), jnp.float32) for _ in range(_NBUF)],
        [pltpu.SemaphoreType.DMA for _ in range(_NBUF)],
        [pltpu.SemaphoreType.DMA for _ in range(_NBUF)],
    ],
    compiler_params=pltpu.CompilerParams(use_tc_tiling_on_sc=False),
)
def _gather_kernel(idx_hbm, table_hbm, rows_hbm, idx_v, gbuf, sem_g, sem_w):
    wid = lax.axis_index("s") * _NC + lax.axis_index("c")
    b0 = pl.multiple_of(wid * _BLK, _BLK)
    pltpu.sync_copy(idx_hbm.at[:, pl.ds(b0, _BLK)], idx_v)

    def fire_gather(t, b):
        pltpu.async_copy(table_hbm.at[idx_v.at[t]], gbuf[b], sem_g[b])

    def wait_gather(b):
        pltpu.make_async_copy(table_hbm.at[idx_v.at[0]], gbuf[b], sem_g[b]).wait()

    def fire_writeback(t, b):
        pltpu.async_copy(gbuf[b].at[pl.ds(0, _D), :], rows_hbm.at[t, wid, 0],
                         sem_w[b])
        pltpu.async_copy(gbuf[b].at[pl.ds(_D, _D), :], rows_hbm.at[t, wid, 1],
                         sem_w[b])

    def wait_writeback(b):
        for _ in range(2):
            pltpu.make_async_copy(gbuf[b].at[pl.ds(0, _D), :],
                                  rows_hbm.at[0, 0, 0], sem_w[b]).wait()

    for b in range(_NBUF):
        fire_gather(b, b)
    for b in range(_NBUF):
        wait_gather(b)
        fire_writeback(b, b)

    @pl.loop(1, _NG)
    def _group(gi):
        t0 = gi * _NBUF
        for b in range(_NBUF):
            wait_writeback(b)
            fire_gather(t0 + b, b)
        for b in range(_NBUF):
            wait_gather(b)
            fire_writeback(t0 + b, b)

    for b in range(_NBUF):
        wait_writeback(b)


@functools.partial(
    pl.kernel,
    out_type=jax.ShapeDtypeStruct((_T, _D // 8, _NW, 8, _BLK), jnp.float32),
    mesh=_mesh,
    scratch_types=[
        [pltpu.VMEM((2, _D, _D), jnp.float32) for _ in range(_NBUF)],
        [pltpu.VMEM((_D // 8, 8, _BLK + 1), jnp.float32) for _ in range(_NBUF)],
        [pltpu.SemaphoreType.DMA for _ in range(_NBUF)],
        [pltpu.SemaphoreType.DMA for _ in range(_NBUF)],
    ],
    compiler_params=pltpu.CompilerParams(use_tc_tiling_on_sc=False,
                                         needs_layout_passes=False),
)
def _retile_kernel(rows_hbm, out_hbm, ibuf, tbuf, sem_i, sem_w):
    wid = lax.axis_index("s") * _NC + lax.axis_index("c")

    def fire_read(t, b):
        pltpu.async_copy(rows_hbm.at[t, wid], ibuf[b], sem_i[b])

    def wait_read(b):
        pltpu.make_async_copy(rows_hbm.at[0, 0], ibuf[b], sem_i[b]).wait()

    lane = jnp.arange(16, dtype=jnp.int32)
    er_ids = [(lane >> 3) + 2 * g for g in range(_D // 16)]
    ei_ids = lane & 7

    def transpose(b):
        @pl.loop(0, _BLK, unroll=8)
        def _row(j):
            col = jnp.full((16,), j, dtype=jnp.int32)
            vals = [ibuf[b][j >> 6, j & (_D - 1), pl.ds(g * 16, 16)]
                    for g in range(_D // 16)]
            for g, v in enumerate(vals):
                plsc.store_scatter(tbuf[b], [er_ids[g], ei_ids, col], v)

    def fire_writeback(t, b):
        pltpu.async_copy(tbuf[b].at[:, :, pl.ds(0, _BLK)],
                         out_hbm.at[t, :, wid, :, :], sem_w[b])

    def wait_writeback(b):
        pltpu.make_async_copy(tbuf[b].at[:, :, pl.ds(0, _BLK)],
                              out_hbm.at[0, :, 0, :, :], sem_w[b]).wait()

    for b in range(_NBUF):
        fire_read(b, b)
    for b in range(_NBUF):
        wait_read(b)
        transpose(b)
        fire_writeback(b, b)
        fire_read(b + _NBUF, b)

    @pl.loop(1, _NG - 1)
    def _group(gi):
        t0 = gi * _NBUF
        for b in range(_NBUF):
            wait_read(b)
            wait_writeback(b)
            transpose(b)
            fire_writeback(t0 + b, b)
            fire_read(t0 + b + _NBUF, b)

    t0 = (_NG - 1) * _NBUF
    for b in range(_NBUF):
        wait_read(b)
        wait_writeback(b)
        transpose(b)
        fire_writeback(t0 + b, b)
    for b in range(_NBUF):
        wait_writeback(b)


_VB = 4096
_NVB = 245
_VPAD = _NVB * _VB


def _transpose_body(x_ref, o_ref):
    x = x_ref[...]
    o_ref[...] = jnp.concatenate(
        [x[:, :_VB // 2].T, x[:, _VB // 2:].T], axis=1)


def _transpose_tc(wt):
    return pl.pallas_call(
        _transpose_body,
        out_shape=jax.ShapeDtypeStruct((_VPAD // 2, 2 * _D), jnp.float32),
        grid=(_NVB,),
        in_specs=[pl.BlockSpec((_D, _VB), lambda i: (0, i))],
        out_specs=pl.BlockSpec((_VB // 2, 2 * _D), lambda i: (i, 0)),
    )(wt)


def kernel(input_ids, weight):
    v = input_ids.T.astype(jnp.int32)
    j = v & (_VB - 1)
    idx_t = (v - j) + 2 * (j & (_VB // 2 - 1)) + (j >> 11)
    table = _transpose_tc(weight.T).reshape(_VPAD, _D)
    rows = _gather_kernel(idx_t, table)
    out5 = _retile_kernel(rows)
    out = out5.transpose(2, 4, 0, 1, 3).reshape(_BB, _T, _D)
    return out


# speedup vs baseline: 2.4553x; 1.1710x over previous
import functools

import jax
import jax.numpy as jnp
from jax import lax
from jax.experimental import pallas as pl
from jax.experimental.pallas import tpu as pltpu
from jax.experimental.pallas import tpu_sc as plsc

_NC, _NS = 2, 16
_NW = _NC * _NS
_T = 50
_BB = 4096
_D = 64
_BLK = _BB // _NW
_NBUF = 5
_NG = _T // _NBUF

_mesh = plsc.VectorSubcoreMesh(core_axis_name="c", subcore_axis_name="s")


@functools.partial(
    pl.kernel,
    out_type=jax.ShapeDtypeStruct((_T, _NW, 2, _D, _D), jnp.float32),
    mesh=_mesh,
    scratch_types=[
        pltpu.VMEM((_T, _BLK), jnp.int32),
        [pltpu.VMEM((_BLK, _D), jnp.float32) for _ in range(_NBUF)],
        [pltpu.SemaphoreType.DMA for _ in range(_NBUF)],
        [pltpu.SemaphoreType.DMA for _ in range(_NBUF)],
    ],
    compiler_params=pltpu.CompilerParams(use_tc_tiling_on_sc=False),
)
def _gather_kernel(idx_hbm, table_hbm, rows_hbm, idx_v, gbuf, sem_g, sem_w):
    wid = lax.axis_index("s") * _NC + lax.axis_index("c")
    b0 = pl.multiple_of(wid * _BLK, _BLK)
    pltpu.sync_copy(idx_hbm.at[:, pl.ds(b0, _BLK)], idx_v)

    def fire_gather(t, b):
        pltpu.async_copy(table_hbm.at[idx_v.at[t]], gbuf[b], sem_g[b])

    def wait_gather(b):
        pltpu.make_async_copy(table_hbm.at[idx_v.at[0]], gbuf[b], sem_g[b]).wait()

    def fire_writeback(t, b):
        pltpu.async_copy(gbuf[b].at[pl.ds(0, _D), :], rows_hbm.at[t, wid, 0],
                         sem_w[b])
        pltpu.async_copy(gbuf[b].at[pl.ds(_D, _D), :], rows_hbm.at[t, wid, 1],
                         sem_w[b])

    def wait_writeback(b):
        for _ in range(2):
            pltpu.make_async_copy(gbuf[b].at[pl.ds(0, _D), :],
                                  rows_hbm.at[0, 0, 0], sem_w[b]).wait()

    for b in range(_NBUF):
        fire_gather(b, b)
    for b in range(_NBUF):
        wait_gather(b)
        fire_writeback(b, b)

    @pl.loop(1, _NG)
    def _group(gi):
        t0 = gi * _NBUF
        for b in range(_NBUF):
            wait_writeback(b)
            fire_gather(t0 + b, b)
        for b in range(_NBUF):
            wait_gather(b)
            fire_writeback(t0 + b, b)

    for b in range(_NBUF):
        wait_writeback(b)


@functools.partial(
    pl.kernel,
    out_type=jax.ShapeDtypeStruct((_T, _D // 8, _NW, 8, _BLK), jnp.float32),
    mesh=_mesh,
    scratch_types=[
        [pltpu.VMEM((2, _D, _D), jnp.float32) for _ in range(_NBUF)],
        [pltpu.VMEM((_D // 8, 8, _BLK + 1), jnp.float32) for _ in range(_NBUF)],
        [pltpu.SemaphoreType.DMA for _ in range(_NBUF)],
        [pltpu.SemaphoreType.DMA for _ in range(_NBUF)],
    ],
    compiler_params=pltpu.CompilerParams(use_tc_tiling_on_sc=False,
                                         needs_layout_passes=False),
)
def _retile_kernel(rows_hbm, out_hbm, ibuf, tbuf, sem_i, sem_w):
    wid = lax.axis_index("s") * _NC + lax.axis_index("c")

    def fire_read(t, b):
        pltpu.async_copy(rows_hbm.at[t, wid], ibuf[b], sem_i[b])

    def wait_read(b):
        pltpu.make_async_copy(rows_hbm.at[0, 0], ibuf[b], sem_i[b]).wait()

    lane = jnp.arange(16, dtype=jnp.int32)
    er_ids = [(lane >> 3) + 2 * g for g in range(_D // 16)]
    ei_ids = lane & 7

    def transpose(b):
        @pl.loop(0, _BLK, unroll=8)
        def _row(j):
            col = jnp.full((16,), j, dtype=jnp.int32)
            vals = [ibuf[b][j >> 6, j & (_D - 1), pl.ds(g * 16, 16)]
                    for g in range(_D // 16)]
            for g, v in enumerate(vals):
                plsc.store_scatter(tbuf[b], [er_ids[g], ei_ids, col], v)

    def fire_writeback(t, b):
        pltpu.async_copy(tbuf[b].at[:, :, pl.ds(0, _BLK)],
                         out_hbm.at[t, :, wid, :, :], sem_w[b])

    def wait_writeback(b):
        pltpu.make_async_copy(tbuf[b].at[:, :, pl.ds(0, _BLK)],
                              out_hbm.at[0, :, 0, :, :], sem_w[b]).wait()

    for b in range(_NBUF):
        fire_read(b, b)
    for b in range(_NBUF):
        wait_read(b)
        transpose(b)
        fire_writeback(b, b)
        fire_read(b + _NBUF, b)

    @pl.loop(1, _NG - 1)
    def _group(gi):
        t0 = gi * _NBUF
        for b in range(_NBUF):
            wait_read(b)
            wait_writeback(b)
            transpose(b)
            fire_writeback(t0 + b, b)
            fire_read(t0 + b + _NBUF, b)

    t0 = (_NG - 1) * _NBUF
    for b in range(_NBUF):
        wait_read(b)
        wait_writeback(b)
        transpose(b)
        fire_writeback(t0 + b, b)
    for b in range(_NBUF):
        wait_writeback(b)


_VB = 8192
_NVB = 123
_VPAD = _NVB * _VB


def _transpose_body(x_ref, o_ref):
    x = x_ref[...]
    o_ref[...] = jnp.concatenate(
        [x[:, :_VB // 2].T, x[:, _VB // 2:].T], axis=1)


def _transpose_tc(wt):
    return pl.pallas_call(
        _transpose_body,
        out_shape=jax.ShapeDtypeStruct((_VPAD // 2, 2 * _D), jnp.float32),
        grid=(_NVB,),
        in_specs=[pl.BlockSpec((_D, _VB), lambda i: (0, i))],
        out_specs=pl.BlockSpec((_VB // 2, 2 * _D), lambda i: (i, 0)),
    )(wt)


def kernel(input_ids, weight):
    v = input_ids.T.astype(jnp.int32)
    j = v & (_VB - 1)
    idx_t = (v - j) + 2 * (j & (_VB // 2 - 1)) + (j >> 12)
    table = _transpose_tc(weight.T).reshape(_VPAD, _D)
    rows = _gather_kernel(idx_t, table)
    out5 = _retile_kernel(rows)
    out = out5.transpose(2, 4, 0, 1, 3).reshape(_BB, _T, _D)
    return out


# speedup vs baseline: 2.6778x; 1.0906x over previous
import functools

import jax
import jax.numpy as jnp
from jax import lax
from jax.experimental import pallas as pl
from jax.experimental.pallas import tpu as pltpu
from jax.experimental.pallas import tpu_sc as plsc

_NC, _NS = 2, 16
_NW = _NC * _NS
_T = 50
_BB = 4096
_D = 64
_BLK = _BB // _NW
_NBUF = 5
_NG = _T // _NBUF

_mesh = plsc.VectorSubcoreMesh(core_axis_name="c", subcore_axis_name="s")


@functools.partial(
    pl.kernel,
    out_type=jax.ShapeDtypeStruct((_T, _NW, 2, _D, _D), jnp.float32),
    mesh=_mesh,
    scratch_types=[
        pltpu.VMEM((_T, _BLK), jnp.int32),
        [pltpu.VMEM((_BLK, _D), jnp.float32) for _ in range(_NBUF)],
        [pltpu.SemaphoreType.DMA for _ in range(_NBUF)],
        [pltpu.SemaphoreType.DMA for _ in range(_NBUF)],
    ],
    compiler_params=pltpu.CompilerParams(use_tc_tiling_on_sc=False),
)
def _gather_kernel(idx_hbm, table_hbm, rows_hbm, idx_v, gbuf, sem_g, sem_w):
    wid = lax.axis_index("s") * _NC + lax.axis_index("c")
    b0 = pl.multiple_of(wid * _BLK, _BLK)
    pltpu.sync_copy(idx_hbm.at[:, pl.ds(b0, _BLK)], idx_v)

    def fire_gather(t, b):
        pltpu.async_copy(table_hbm.at[idx_v.at[t]], gbuf[b], sem_g[b])

    def wait_gather(b):
        pltpu.make_async_copy(table_hbm.at[idx_v.at[0]], gbuf[b], sem_g[b]).wait()

    def fire_writeback(t, b):
        pltpu.async_copy(gbuf[b].at[pl.ds(0, _D), :], rows_hbm.at[t, wid, 0],
                         sem_w[b])
        pltpu.async_copy(gbuf[b].at[pl.ds(_D, _D), :], rows_hbm.at[t, wid, 1],
                         sem_w[b])

    def wait_writeback(b):
        for _ in range(2):
            pltpu.make_async_copy(gbuf[b].at[pl.ds(0, _D), :],
                                  rows_hbm.at[0, 0, 0], sem_w[b]).wait()

    for b in range(_NBUF):
        fire_gather(b, b)
    for b in range(_NBUF):
        wait_gather(b)
        fire_writeback(b, b)

    @pl.loop(1, _NG)
    def _group(gi):
        t0 = gi * _NBUF
        for b in range(_NBUF):
            wait_writeback(b)
            fire_gather(t0 + b, b)
        for b in range(_NBUF):
            wait_gather(b)
            fire_writeback(t0 + b, b)

    for b in range(_NBUF):
        wait_writeback(b)


@functools.partial(
    pl.kernel,
    out_type=jax.ShapeDtypeStruct((_T, _D // 8, _NW, 8, _BLK), jnp.float32),
    mesh=_mesh,
    scratch_types=[
        [pltpu.VMEM((2, _D, _D), jnp.float32) for _ in range(_NBUF)],
        [pltpu.VMEM((_D // 8, 8, _BLK + 1), jnp.float32) for _ in range(_NBUF)],
        [pltpu.SemaphoreType.DMA for _ in range(_NBUF)],
        [pltpu.SemaphoreType.DMA for _ in range(_NBUF)],
    ],
    compiler_params=pltpu.CompilerParams(use_tc_tiling_on_sc=False,
                                         needs_layout_passes=False),
)
def _retile_kernel(rows_hbm, out_hbm, ibuf, tbuf, sem_i, sem_w):
    wid = lax.axis_index("s") * _NC + lax.axis_index("c")

    def fire_read(t, b):
        pltpu.async_copy(rows_hbm.at[t, wid], ibuf[b], sem_i[b])

    def wait_read(b):
        pltpu.make_async_copy(rows_hbm.at[0, 0], ibuf[b], sem_i[b]).wait()

    lane = jnp.arange(16, dtype=jnp.int32)
    er_ids = [(lane >> 3) + 2 * g for g in range(_D // 16)]
    ei_ids = lane & 7

    def transpose(b):
        @pl.loop(0, _BLK, unroll=8)
        def _row(j):
            col = jnp.full((16,), j, dtype=jnp.int32)
            vals = [ibuf[b][j >> 6, j & (_D - 1), pl.ds(g * 16, 16)]
                    for g in range(_D // 16)]
            for g, v in enumerate(vals):
                plsc.store_scatter(tbuf[b], [er_ids[g], ei_ids, col], v)

    def fire_writeback(t, b):
        pltpu.async_copy(tbuf[b].at[:, :, pl.ds(0, _BLK)],
                         out_hbm.at[t, :, wid, :, :], sem_w[b])

    def wait_writeback(b):
        pltpu.make_async_copy(tbuf[b].at[:, :, pl.ds(0, _BLK)],
                              out_hbm.at[0, :, 0, :, :], sem_w[b]).wait()

    for b in range(_NBUF):
        fire_read(b, b)
    for b in range(_NBUF):
        wait_read(b)
        transpose(b)
        fire_writeback(b, b)
        fire_read(b + _NBUF, b)

    @pl.loop(1, _NG - 1)
    def _group(gi):
        t0 = gi * _NBUF
        for b in range(_NBUF):
            wait_read(b)
            wait_writeback(b)
            transpose(b)
            fire_writeback(t0 + b, b)
            fire_read(t0 + b + _NBUF, b)

    t0 = (_NG - 1) * _NBUF
    for b in range(_NBUF):
        wait_read(b)
        wait_writeback(b)
        transpose(b)
        fire_writeback(t0 + b, b)
    for b in range(_NBUF):
        wait_writeback(b)


_VB = 16384
_NVB = 62
_VPAD = _NVB * _VB


def _transpose_body(x_ref, o_ref):
    x = x_ref[...]
    o_ref[...] = jnp.concatenate(
        [x[:, :_VB // 2].T, x[:, _VB // 2:].T], axis=1)


def _transpose_tc(wt):
    return pl.pallas_call(
        _transpose_body,
        out_shape=jax.ShapeDtypeStruct((_VPAD // 2, 2 * _D), jnp.float32),
        grid=(_NVB,),
        in_specs=[pl.BlockSpec((_D, _VB), lambda i: (0, i))],
        out_specs=pl.BlockSpec((_VB // 2, 2 * _D), lambda i: (i, 0)),
    )(wt)


def kernel(input_ids, weight):
    v = input_ids.T.astype(jnp.int32)
    j = v & (_VB - 1)
    idx_t = (v - j) + 2 * (j & (_VB // 2 - 1)) + (j >> 13)
    table = _transpose_tc(weight.T).reshape(_VPAD, _D)
    rows = _gather_kernel(idx_t, table)
    out5 = _retile_kernel(rows)
    out = out5.transpose(2, 4, 0, 1, 3).reshape(_BB, _T, _D)
    return out


# speedup vs baseline: 2.7850x; 1.0400x over previous
import functools

import jax
import jax.numpy as jnp
from jax import lax
from jax.experimental import pallas as pl
from jax.experimental.pallas import tpu as pltpu
from jax.experimental.pallas import tpu_sc as plsc

_NC, _NS = 2, 16
_NW = _NC * _NS
_T = 50
_BB = 4096
_D = 64
_BLK = _BB // _NW
_NBUF = 5
_NG = _T // _NBUF

_mesh = plsc.VectorSubcoreMesh(core_axis_name="c", subcore_axis_name="s")


@functools.partial(
    pl.kernel,
    out_type=jax.ShapeDtypeStruct((_T, _NW, 2, _D, _D), jnp.float32),
    mesh=_mesh,
    scratch_types=[
        pltpu.VMEM((_T, _BLK), jnp.int32),
        [pltpu.VMEM((_BLK, _D), jnp.float32) for _ in range(_NBUF)],
        [pltpu.SemaphoreType.DMA for _ in range(_NBUF)],
        [pltpu.SemaphoreType.DMA for _ in range(_NBUF)],
    ],
    compiler_params=pltpu.CompilerParams(use_tc_tiling_on_sc=False),
)
def _gather_kernel(idx_hbm, table_hbm, rows_hbm, idx_v, gbuf, sem_g, sem_w):
    wid = lax.axis_index("s") * _NC + lax.axis_index("c")
    b0 = pl.multiple_of(wid * _BLK, _BLK)
    pltpu.sync_copy(idx_hbm.at[:, pl.ds(b0, _BLK)], idx_v)

    def fire_gather(t, b):
        pltpu.async_copy(table_hbm.at[idx_v.at[t]], gbuf[b], sem_g[b])

    def wait_gather(b):
        pltpu.make_async_copy(table_hbm.at[idx_v.at[0]], gbuf[b], sem_g[b]).wait()

    def fire_writeback(t, b):
        pltpu.async_copy(gbuf[b].at[pl.ds(0, _D), :], rows_hbm.at[t, wid, 0],
                         sem_w[b])
        pltpu.async_copy(gbuf[b].at[pl.ds(_D, _D), :], rows_hbm.at[t, wid, 1],
                         sem_w[b])

    def wait_writeback(b):
        for _ in range(2):
            pltpu.make_async_copy(gbuf[b].at[pl.ds(0, _D), :],
                                  rows_hbm.at[0, 0, 0], sem_w[b]).wait()

    for b in range(_NBUF):
        fire_gather(b, b)
    for b in range(_NBUF):
        wait_gather(b)
        fire_writeback(b, b)

    @pl.loop(1, _NG)
    def _group(gi):
        t0 = gi * _NBUF
        for b in range(_NBUF):
            wait_writeback(b)
            fire_gather(t0 + b, b)
        for b in range(_NBUF):
            wait_gather(b)
            fire_writeback(t0 + b, b)

    for b in range(_NBUF):
        wait_writeback(b)


@functools.partial(
    pl.kernel,
    out_type=jax.ShapeDtypeStruct((_T, _D // 8, _NW, 8, _BLK), jnp.float32),
    mesh=_mesh,
    scratch_types=[
        [pltpu.VMEM((2, _D, _D), jnp.float32) for _ in range(_NBUF)],
        [pltpu.VMEM((_D // 8, 8, _BLK + 1), jnp.float32) for _ in range(_NBUF)],
        [pltpu.SemaphoreType.DMA for _ in range(_NBUF)],
        [pltpu.SemaphoreType.DMA for _ in range(_NBUF)],
    ],
    compiler_params=pltpu.CompilerParams(use_tc_tiling_on_sc=False,
                                         needs_layout_passes=False),
)
def _retile_kernel(rows_hbm, out_hbm, ibuf, tbuf, sem_i, sem_w):
    wid = lax.axis_index("s") * _NC + lax.axis_index("c")

    def fire_read(t, b):
        pltpu.async_copy(rows_hbm.at[t, wid], ibuf[b], sem_i[b])

    def wait_read(b):
        pltpu.make_async_copy(rows_hbm.at[0, 0], ibuf[b], sem_i[b]).wait()

    lane = jnp.arange(16, dtype=jnp.int32)
    er_ids = [(lane >> 3) + 2 * g for g in range(_D // 16)]
    ei_ids = lane & 7

    def transpose(b):
        @pl.loop(0, _BLK, unroll=8)
        def _row(j):
            col = jnp.full((16,), j, dtype=jnp.int32)
            vals = [ibuf[b][j >> 6, j & (_D - 1), pl.ds(g * 16, 16)]
                    for g in range(_D // 16)]
            for g, v in enumerate(vals):
                plsc.store_scatter(tbuf[b], [er_ids[g], ei_ids, col], v)

    def fire_writeback(t, b):
        pltpu.async_copy(tbuf[b].at[:, :, pl.ds(0, _BLK)],
                         out_hbm.at[t, :, wid, :, :], sem_w[b])

    def wait_writeback(b):
        pltpu.make_async_copy(tbuf[b].at[:, :, pl.ds(0, _BLK)],
                              out_hbm.at[0, :, 0, :, :], sem_w[b]).wait()

    for b in range(_NBUF):
        fire_read(b, b)
    for b in range(_NBUF):
        wait_read(b)
        transpose(b)
        fire_writeback(b, b)
        fire_read(b + _NBUF, b)

    @pl.loop(1, _NG - 1)
    def _group(gi):
        t0 = gi * _NBUF
        for b in range(_NBUF):
            wait_read(b)
            wait_writeback(b)
            transpose(b)
            fire_writeback(t0 + b, b)
            fire_read(t0 + b + _NBUF, b)

    t0 = (_NG - 1) * _NBUF
    for b in range(_NBUF):
        wait_read(b)
        wait_writeback(b)
        transpose(b)
        fire_writeback(t0 + b, b)
    for b in range(_NBUF):
        wait_writeback(b)


_VB = 32768
_NVB = 31
_VPAD = _NVB * _VB


def _transpose_body(x_ref, o_ref):
    x = x_ref[...]
    o_ref[...] = jnp.concatenate(
        [x[:, :_VB // 2].T, x[:, _VB // 2:].T], axis=1)


def _transpose_tc(wt):
    return pl.pallas_call(
        _transpose_body,
        out_shape=jax.ShapeDtypeStruct((_VPAD // 2, 2 * _D), jnp.float32),
        grid=(_NVB,),
        in_specs=[pl.BlockSpec((_D, _VB), lambda i: (0, i))],
        out_specs=pl.BlockSpec((_VB // 2, 2 * _D), lambda i: (i, 0)),
    )(wt)


def kernel(input_ids, weight):
    v = input_ids.T.astype(jnp.int32)
    j = v & (_VB - 1)
    idx_t = (v - j) + 2 * (j & (_VB // 2 - 1)) + (j >> 14)
    table = _transpose_tc(weight.T).reshape(_VPAD, _D)
    rows = _gather_kernel(idx_t, table)
    out5 = _retile_kernel(rows)
    out = out5.transpose(2, 4, 0, 1, 3).reshape(_BB, _T, _D)
    return out
